# Initial kernel scaffold; baseline (speedup 1.0000x reference)
import functools

import jax
import jax.numpy as jnp
from jax import lax
from jax.experimental import pallas as pl
from jax.experimental.pallas import tpu as pltpu
from jax.experimental.pallas import tpu_sc as plsc

N = 10000
D = 128
E = 320000
NC = 2
NS = 16
TILES = NC * NS
EPT = E // TILES
C = 80
NCHUNK = EPT // C
RPT = N // NS


def _prep_body(x_ref, beta_ref, xn_ref, norm_ref, sw_ref):
    x = x_ref[...]
    n2 = jnp.sum(x * x, axis=1, keepdims=True)
    nrm = jnp.sqrt(n2)
    xn = x / jnp.maximum(nrm, 1e-12)
    xn_ref[...] = xn
    norm_ref[...] = nrm
    sd = jnp.sum(xn * xn, axis=1, keepdims=True)
    sw_ref[...] = jnp.exp(beta_ref[...] * sd)


def _prep(x, beta2d):
    return pl.pallas_call(
        _prep_body,
        out_shape=(
            jax.ShapeDtypeStruct((N, D), jnp.float32),
            jax.ShapeDtypeStruct((N, 1), jnp.float32),
            jax.ShapeDtypeStruct((N, 1), jnp.float32),
        ),
    )(x, beta2d)


def _edge_body(xn_hbm, srci_hbm, dsti_hbm, normv_hbm, betav_hbm,
               num_out, den_out,
               xs, xd, wrow, srcv, dstv, normt, tmp, betv, wnv,
               num_sh, den_sh, sem1, sem2):
    cidx = lax.axis_index("c")
    sidx = lax.axis_index("s")
    wid = cidx * NS + sidx
    nrow0 = sidx * RPT

    pltpu.sync_copy(srci_hbm.at[wid], srcv)
    pltpu.sync_copy(dsti_hbm.at[wid], dstv)
    pltpu.sync_copy(normv_hbm, normt)
    pltpu.sync_copy(betav_hbm, betv)

    z16 = jnp.zeros((16,), jnp.float32)

    @pl.loop(0, C)
    def _(r):
        for dd in range(D // 16):
            xs[r, pl.ds(dd * 16, 16)] = z16
        wrow[r, pl.ds(0, 16)] = z16

    @pl.loop(0, RPT // C)
    def _(k):
        pltpu.sync_copy(xs, num_sh.at[pl.ds(nrow0 + k * C, C)])
        pltpu.sync_copy(wrow, den_sh.at[pl.ds(nrow0 + k * C, C)])
    rem = RPT - (RPT // C) * C
    pltpu.sync_copy(xs.at[pl.ds(0, rem)],
                    num_sh.at[pl.ds(nrow0 + RPT - rem, rem)])
    pltpu.sync_copy(wrow.at[pl.ds(0, rem)],
                    den_sh.at[pl.ds(nrow0 + RPT - rem, rem)])

    plsc.subcore_barrier()

    iota16 = lax.iota(jnp.int32, 16)
    bet = betv[pl.ds(0, 16)]

    @pl.loop(0, NCHUNK)
    def _(j):
        cp1 = pltpu.async_copy(xn_hbm.at[srcv.at[j]], xs, sem1)
        cp2 = pltpu.async_copy(xn_hbm.at[dstv.at[j]], xd, sem2)
        cp1.wait()
        cp2.wait()

        @pl.loop(0, C // 16)
        def _(g):
            r0 = g * 16
            src16 = srcv[j, pl.ds(r0, 16)]
            normsrc = plsc.load_gather(normt, [src16])
            for e in range(16):
                v = xs[r0 + e, pl.ds(0, 16)] * xd[r0 + e, pl.ds(0, 16)]
                for dd in range(1, D // 16):
                    v = v + (xs[r0 + e, pl.ds(dd * 16, 16)]
                             * xd[r0 + e, pl.ds(dd * 16, 16)])
                tmp[e, pl.ds(0, 16)] = v
            dots = plsc.load_gather(tmp, [iota16, jnp.zeros((16,), jnp.int32)])
            for dd in range(1, 16):
                dots = dots + plsc.load_gather(
                    tmp, [iota16, jnp.full((16,), dd, jnp.int32)])
            wden = jnp.exp(dots * bet)
            wnum = wden * normsrc
            plsc.store_scatter(
                wrow, [iota16 + r0, jnp.zeros((16,), jnp.int32)], wden)
            wnv[pl.ds(0, 16)] = wnum
            for e in range(16):
                sca = plsc.load_gather(wnv, [jnp.full((16,), e, jnp.int32)])
                for dd in range(D // 16):
                    xs[r0 + e, pl.ds(dd * 16, 16)] = (
                        xs[r0 + e, pl.ds(dd * 16, 16)] * sca)

        pltpu.sync_copy(xs, num_sh.at[dstv.at[j]], add=True)
        pltpu.sync_copy(wrow, den_sh.at[dstv.at[j]], add=True)

    plsc.subcore_barrier()

    pltpu.sync_copy(num_sh.at[pl.ds(nrow0, RPT)],
                    num_out.at[pl.ds(cidx * N + nrow0, RPT)])
    pltpu.sync_copy(den_sh.at[pl.ds(nrow0, RPT)],
                    den_out.at[pl.ds(cidx * N + nrow0, RPT)])


def _edge_stage(xn, srci, dsti, normv, betav):
    mesh = plsc.VectorSubcoreMesh(core_axis_name="c", subcore_axis_name="s")
    kern = pl.kernel(
        _edge_body,
        out_type=(
            jax.ShapeDtypeStruct((NC * N, D), jnp.float32),
            jax.ShapeDtypeStruct((NC * N, 16), jnp.float32),
        ),
        mesh=mesh,
        scratch_types=[
            pltpu.VMEM((C, D), jnp.float32),
            pltpu.VMEM((C, D), jnp.float32),
            pltpu.VMEM((C, 16), jnp.float32),
            pltpu.VMEM((NCHUNK, C), jnp.int32),
            pltpu.VMEM((NCHUNK, C), jnp.int32),
            pltpu.VMEM((N,), jnp.float32),
            pltpu.VMEM((16, 16), jnp.float32),
            pltpu.VMEM((16,), jnp.float32),
            pltpu.VMEM((16,), jnp.float32),
            pltpu.VMEM_SHARED((N, D), jnp.float32),
            pltpu.VMEM_SHARED((N, 16), jnp.float32),
            pltpu.SemaphoreType.DMA,
            pltpu.SemaphoreType.DMA,
        ],
    )
    return kern(xn, srci, dsti, normv, betav)


def _lstm_body(num_ref, den_ref, sw_ref, x_ref, h_ref, c_ref,
               wih_ref, whh_ref, h1_ref, c1_ref):
    sw = sw_ref[...]
    num = num_ref[0] + num_ref[1] + sw * x_ref[...]
    den = den_ref[0, :, 0:1] + den_ref[1, :, 0:1] + sw
    xb = jnp.tanh(num / jnp.maximum(den, 1e-16))
    dn = (((1,), (1,)), ((), ()))
    gates = lax.dot_general(xb, wih_ref[...], dn,
                            preferred_element_type=jnp.float32,
                            precision=lax.Precision.HIGHEST)
    gates = gates + lax.dot_general(h_ref[...], whh_ref[...], dn,
                                    preferred_element_type=jnp.float32,
                                    precision=lax.Precision.HIGHEST)
    ii = jax.nn.sigmoid(gates[:, 0:D])
    ff = jax.nn.sigmoid(gates[:, D:2 * D])
    gg = jnp.tanh(gates[:, 2 * D:3 * D])
    oo = jax.nn.sigmoid(gates[:, 3 * D:4 * D])
    c1 = ff * c_ref[...] + ii * gg
    h1_ref[...] = oo * jnp.tanh(c1)
    c1_ref[...] = c1


def _lstm_stage(num2, den2, sw, x, h0, c0, W_ih, W_hh):
    nb = 10
    blk = N // nb
    return pl.pallas_call(
        _lstm_body,
        grid=(nb,),
        in_specs=[
            pl.BlockSpec((2, blk, D), lambda i: (0, i, 0)),
            pl.BlockSpec((2, blk, 16), lambda i: (0, i, 0)),
            pl.BlockSpec((blk, 1), lambda i: (i, 0)),
            pl.BlockSpec((blk, D), lambda i: (i, 0)),
            pl.BlockSpec((blk, D), lambda i: (i, 0)),
            pl.BlockSpec((blk, D), lambda i: (i, 0)),
            pl.BlockSpec((4 * D, D), lambda i: (0, 0)),
            pl.BlockSpec((4 * D, D), lambda i: (0, 0)),
        ],
        out_specs=[
            pl.BlockSpec((blk, D), lambda i: (i, 0)),
            pl.BlockSpec((blk, D), lambda i: (i, 0)),
        ],
        out_shape=(
            jax.ShapeDtypeStruct((N, D), jnp.float32),
            jax.ShapeDtypeStruct((N, D), jnp.float32),
        ),
    )(num2, den2, sw, x, h0, c0, W_ih, W_hh)


def kernel(x, edge_index, h, c, beta, W_ih, W_hh):
    beta2d = jnp.reshape(beta.astype(jnp.float32), (1, 1))
    xn, normv, sw = _prep(x, beta2d)

    srci = jnp.reshape(edge_index[0], (TILES, NCHUNK, C))
    dsti = jnp.reshape(edge_index[1], (TILES, NCHUNK, C))
    betav = jnp.broadcast_to(jnp.reshape(beta.astype(jnp.float32), (1,)), (16,))
    num2, den2 = _edge_stage(xn, srci, dsti, jnp.reshape(normv, (N,)), betav)

    h1, c1 = _lstm_stage(
        jnp.reshape(num2, (NC, N, D)), jnp.reshape(den2, (NC, N, 16)),
        sw, x, h[0], c[0], W_ih, W_hh)
    return (h1, h1[None, :, :], c1[None, :, :])


# speedup vs baseline: 12.7380x; 12.7380x over previous
import dataclasses
import functools

import jax
import jax.numpy as jnp
from jax import lax
from jax.experimental import pallas as pl
from jax.experimental.pallas import tpu as pltpu
from jax.experimental.pallas import tpu_sc as plsc

N = 10000
D = 128
E = 320000
NC = 2
NS = 16
TILES = NC * NS
EPT = E // TILES
C = 80
NCHUNK = EPT // C
RPT = 624
TAIL = N - NS * RPT


def _prep_body(x_ref, beta_ref, xn_ref, norm_ref, sw_ref):
    x = x_ref[...]
    n2 = jnp.sum(x * x, axis=1, keepdims=True)
    nrm = jnp.sqrt(n2)
    xn = x / jnp.maximum(nrm, 1e-12)
    xn_ref[...] = xn
    norm_ref[...] = nrm
    sd = jnp.sum(xn * xn, axis=1, keepdims=True)
    sw_ref[...] = jnp.exp(beta_ref[...] * sd)


def _prep(x, beta2d):
    return pl.pallas_call(
        _prep_body,
        out_shape=(
            jax.ShapeDtypeStruct((N, D), jnp.float32),
            jax.ShapeDtypeStruct((N, 1), jnp.float32),
            jax.ShapeDtypeStruct((N, 1), jnp.float32),
        ),
    )(x, beta2d)


def _edge_body(xn_hbm, norm2d_hbm, srci_hbm, dsti_hbm, betav_hbm,
               num_out, den_out,
               xs, xd, wrow, normc, srcv, dstv, tmp, betv, wnv,
               num_sh, den_sh, sem1, sem2, sem3):
    cidx = lax.axis_index("c")
    sidx = lax.axis_index("s")
    wid = cidx * NS + sidx
    nrow0 = sidx * RPT
    ebase = wid * EPT

    pltpu.sync_copy(betav_hbm, betv)

    z16 = jnp.zeros((16,), jnp.float32)

    @pl.loop(0, C)
    def _(r):
        for dd in range(D // 16):
            xd[r, pl.ds(dd * 16, 16)] = z16
        wrow[r, pl.ds(0, 16)] = z16

    @pl.loop(0, RPT // C)
    def _(k):
        pltpu.sync_copy(xd, num_sh.at[pl.ds(nrow0 + k * C, C)])
        pltpu.sync_copy(wrow, den_sh.at[pl.ds(nrow0 + k * C, C)])
    rem = RPT - (RPT // C) * C
    pltpu.sync_copy(xd.at[pl.ds(0, rem)],
                    num_sh.at[pl.ds(nrow0 + RPT - rem, rem)])
    pltpu.sync_copy(wrow.at[pl.ds(0, rem)],
                    den_sh.at[pl.ds(nrow0 + RPT - rem, rem)])

    @pl.when(sidx == 0)
    def _():
        pltpu.sync_copy(xd.at[pl.ds(0, TAIL)],
                        num_sh.at[pl.ds(NS * RPT, TAIL)])
        pltpu.sync_copy(wrow.at[pl.ds(0, TAIL)],
                        den_sh.at[pl.ds(NS * RPT, TAIL)])

    plsc.subcore_barrier()

    iota16 = lax.iota(jnp.int32, 16)
    bet = betv[pl.ds(0, 16)]

    @pl.loop(0, NCHUNK)
    def _(j):
        eb = ebase + j * C
        pltpu.sync_copy(srci_hbm.at[pl.ds(eb, C)], srcv)
        pltpu.sync_copy(dsti_hbm.at[pl.ds(eb, C)], dstv)
        cp1 = pltpu.async_copy(xn_hbm.at[srcv], xs, sem1)
        cp2 = pltpu.async_copy(xn_hbm.at[dstv], xd, sem2)
        cp3 = pltpu.async_copy(norm2d_hbm.at[srcv], normc, sem3)
        cp1.wait()
        cp2.wait()
        cp3.wait()

        @pl.loop(0, C // 16)
        def _(g):
            r0 = g * 16
            normsrc = plsc.load_gather(
                normc, [iota16 + r0, jnp.zeros((16,), jnp.int32)])
            for e in range(16):
                v = xs[r0 + e, pl.ds(0, 16)] * xd[r0 + e, pl.ds(0, 16)]
                for dd in range(1, D // 16):
                    v = v + (xs[r0 + e, pl.ds(dd * 16, 16)]
                             * xd[r0 + e, pl.ds(dd * 16, 16)])
                tmp[e, pl.ds(0, 16)] = v
            dots = plsc.load_gather(tmp, [iota16, jnp.zeros((16,), jnp.int32)])
            for dd in range(1, 16):
                dots = dots + plsc.load_gather(
                    tmp, [iota16, jnp.full((16,), dd, jnp.int32)])
            wden = jnp.exp(dots * bet)
            wnum = wden * normsrc
            plsc.store_scatter(
                wrow, [iota16 + r0, jnp.zeros((16,), jnp.int32)], wden)
            wnv[pl.ds(0, 16)] = wnum
            for e in range(16):
                sca = plsc.load_gather(wnv, [jnp.full((16,), e, jnp.int32)])
                for dd in range(D // 16):
                    xd[r0 + e, pl.ds(dd * 16, 16)] = (
                        xs[r0 + e, pl.ds(dd * 16, 16)] * sca)

        pltpu.sync_copy(xd, num_sh.at[dstv], add=True)
        pltpu.sync_copy(wrow, den_sh.at[dstv], add=True)

    plsc.subcore_barrier()

    pltpu.sync_copy(num_sh.at[pl.ds(nrow0, RPT)],
                    num_out.at[pl.ds(cidx * N + nrow0, RPT)])
    pltpu.sync_copy(den_sh.at[pl.ds(nrow0, RPT)],
                    den_out.at[pl.ds(cidx * N + nrow0, RPT)])

    @pl.when(sidx == 0)
    def _():
        pltpu.sync_copy(num_sh.at[pl.ds(NS * RPT, TAIL)],
                        num_out.at[pl.ds(cidx * N + NS * RPT, TAIL)])
        pltpu.sync_copy(den_sh.at[pl.ds(NS * RPT, TAIL)],
                        den_out.at[pl.ds(cidx * N + NS * RPT, TAIL)])


def _edge_stage(xn, norm2d, srci, dsti, betav):
    mesh = plsc.VectorSubcoreMesh(core_axis_name="c", subcore_axis_name="s")
    cp = pltpu.CompilerParams()
    if "needs_layout_passes" in pltpu.CompilerParams.__dataclass_fields__:
        cp = dataclasses.replace(cp, needs_layout_passes=False)
    if "use_tc_tiling_on_sc" in pltpu.CompilerParams.__dataclass_fields__:
        cp = dataclasses.replace(cp, use_tc_tiling_on_sc=False)
    kern = pl.kernel(
        _edge_body,
        compiler_params=cp,
        out_type=(
            jax.ShapeDtypeStruct((NC * N, D), jnp.float32),
            jax.ShapeDtypeStruct((NC * N, 16), jnp.float32),
        ),
        mesh=mesh,
        scratch_types=[
            pltpu.VMEM((C, D), jnp.float32),
            pltpu.VMEM((C, D), jnp.float32),
            pltpu.VMEM((C, 16), jnp.float32),
            pltpu.VMEM((C, 16), jnp.float32),
            pltpu.VMEM((C,), jnp.int32),
            pltpu.VMEM((C,), jnp.int32),
            pltpu.VMEM((16, 16), jnp.float32),
            pltpu.VMEM((16,), jnp.float32),
            pltpu.VMEM((16,), jnp.float32),
            pltpu.VMEM_SHARED((N, D), jnp.float32),
            pltpu.VMEM_SHARED((N, 16), jnp.float32),
            pltpu.SemaphoreType.DMA,
            pltpu.SemaphoreType.DMA,
            pltpu.SemaphoreType.DMA,
        ],
    )
    return kern(xn, norm2d, srci, dsti, betav)


def _lstm_body(num_ref, den_ref, sw_ref, x_ref, h_ref, c_ref,
               wih_ref, whh_ref, h1_ref, c1_ref):
    sw = sw_ref[...]
    num = num_ref[0] + num_ref[1] + sw * x_ref[...]
    den = den_ref[0, :, 0:1] + den_ref[1, :, 0:1] + sw
    xb = jnp.tanh(num / jnp.maximum(den, 1e-16))
    dn = (((1,), (1,)), ((), ()))
    gates = lax.dot_general(xb, wih_ref[...], dn,
                            preferred_element_type=jnp.float32,
                            precision=lax.Precision.HIGHEST)
    gates = gates + lax.dot_general(h_ref[...], whh_ref[...], dn,
                                    preferred_element_type=jnp.float32,
                                    precision=lax.Precision.HIGHEST)
    ii = jax.nn.sigmoid(gates[:, 0:D])
    ff = jax.nn.sigmoid(gates[:, D:2 * D])
    gg = jnp.tanh(gates[:, 2 * D:3 * D])
    oo = jax.nn.sigmoid(gates[:, 3 * D:4 * D])
    c1 = ff * c_ref[...] + ii * gg
    h1_ref[...] = oo * jnp.tanh(c1)
    c1_ref[...] = c1


def _lstm_stage(num2, den2, sw, x, h0, c0, W_ih, W_hh):
    nb = 10
    blk = N // nb
    return pl.pallas_call(
        _lstm_body,
        grid=(nb,),
        in_specs=[
            pl.BlockSpec((2, blk, D), lambda i: (0, i, 0)),
            pl.BlockSpec((2, blk, 16), lambda i: (0, i, 0)),
            pl.BlockSpec((blk, 1), lambda i: (i, 0)),
            pl.BlockSpec((blk, D), lambda i: (i, 0)),
            pl.BlockSpec((blk, D), lambda i: (i, 0)),
            pl.BlockSpec((blk, D), lambda i: (i, 0)),
            pl.BlockSpec((4 * D, D), lambda i: (0, 0)),
            pl.BlockSpec((4 * D, D), lambda i: (0, 0)),
        ],
        out_specs=[
            pl.BlockSpec((blk, D), lambda i: (i, 0)),
            pl.BlockSpec((blk, D), lambda i: (i, 0)),
        ],
        out_shape=(
            jax.ShapeDtypeStruct((N, D), jnp.float32),
            jax.ShapeDtypeStruct((N, D), jnp.float32),
        ),
    )(num2, den2, sw, x, h0, c0, W_ih, W_hh)


def kernel(x, edge_index, h, c, beta, W_ih, W_hh):
    beta2d = jnp.reshape(beta.astype(jnp.float32), (1, 1))
    xn, normv, sw = _prep(x, beta2d)
    norm2d = jnp.broadcast_to(normv, (N, 16))

    betav = jnp.broadcast_to(jnp.reshape(beta.astype(jnp.float32), (1,)), (16,))
    num2, den2 = _edge_stage(xn, norm2d, edge_index[0], edge_index[1], betav)

    h1, c1 = _lstm_stage(
        jnp.reshape(num2, (NC, N, D)), jnp.reshape(den2, (NC, N, 16)),
        sw, x, h[0], c[0], W_ih, W_hh)
    return (h1, h1[None, :, :], c1[None, :, :])


# speedup vs baseline: 13.3899x; 1.0512x over previous
import dataclasses
import functools

import jax
import jax.numpy as jnp
from jax import lax
from jax.experimental import pallas as pl
from jax.experimental.pallas import tpu as pltpu
from jax.experimental.pallas import tpu_sc as plsc

N = 10000
D = 128
E = 320000
NC = 2
NS = 16
TILES = NC * NS
EPT = E // TILES
C = 80
NCHUNK = EPT // C
RPT = 624
TAIL = N - NS * RPT


def _prep_body(x_ref, beta_ref, xn_ref, norm_ref, sw_ref):
    x = x_ref[...]
    n2 = jnp.sum(x * x, axis=1, keepdims=True)
    nrm = jnp.sqrt(n2)
    xn = x / jnp.maximum(nrm, 1e-12)
    xn_ref[...] = xn
    norm_ref[...] = nrm
    sd = jnp.sum(xn * xn, axis=1, keepdims=True)
    sw_ref[...] = jnp.exp(beta_ref[...] * sd)


def _prep(x, beta2d):
    return pl.pallas_call(
        _prep_body,
        out_shape=(
            jax.ShapeDtypeStruct((N, D), jnp.float32),
            jax.ShapeDtypeStruct((N, 1), jnp.float32),
            jax.ShapeDtypeStruct((N, 1), jnp.float32),
        ),
    )(x, beta2d)


def _edge_body(xn_hbm, norm2d_hbm, srci_hbm, dsti_hbm, betav_hbm,
               num_out, den_out,
               xs, xd, wrow, normc, srcv, dstv, tmp, betv, wnv,
               num_sh, den_sh, semi, semg, sems):
    cidx = lax.axis_index("c")
    sidx = lax.axis_index("s")
    wid = cidx * NS + sidx
    nrow0 = sidx * RPT
    ebase = wid * EPT

    pltpu.sync_copy(betav_hbm, betv)

    z16 = jnp.zeros((16,), jnp.float32)

    @pl.loop(0, C)
    def _(r):
        for dd in range(D // 16):
            xd[r, pl.ds(dd * 16, 16)] = z16
        wrow[r, pl.ds(0, 16)] = z16

    @pl.loop(0, RPT // C)
    def _(k):
        pltpu.sync_copy(xd, num_sh.at[pl.ds(nrow0 + k * C, C)])
        pltpu.sync_copy(wrow, den_sh.at[pl.ds(nrow0 + k * C, C)])
    rem = RPT - (RPT // C) * C
    pltpu.sync_copy(xd.at[pl.ds(0, rem)],
                    num_sh.at[pl.ds(nrow0 + RPT - rem, rem)])
    pltpu.sync_copy(wrow.at[pl.ds(0, rem)],
                    den_sh.at[pl.ds(nrow0 + RPT - rem, rem)])

    @pl.when(sidx == 0)
    def _():
        pltpu.sync_copy(xd.at[pl.ds(0, TAIL)],
                        num_sh.at[pl.ds(NS * RPT, TAIL)])
        pltpu.sync_copy(wrow.at[pl.ds(0, TAIL)],
                        den_sh.at[pl.ds(NS * RPT, TAIL)])

    plsc.subcore_barrier()

    iota16 = lax.iota(jnp.int32, 16)
    bet = betv[pl.ds(0, 16)]

    @pl.loop(0, NCHUNK)
    def _(j):
        eb = ebase + j * C
        ci1 = pltpu.async_copy(srci_hbm.at[pl.ds(eb, C)], srcv, semi)
        ci2 = pltpu.async_copy(dsti_hbm.at[pl.ds(eb, C)], dstv, semi)
        ci1.wait()
        ci2.wait()
        cp1 = pltpu.async_copy(xn_hbm.at[srcv], xs, semg)
        cp2 = pltpu.async_copy(xn_hbm.at[dstv], xd, semg)
        cp3 = pltpu.async_copy(norm2d_hbm.at[srcv], normc, semg)
        cp1.wait()
        cp2.wait()
        cp3.wait()

        @plsc.parallel_loop(0, C // 16)
        def _(g):
            r0 = g * 16
            normsrc = plsc.load_gather(
                normc, [iota16 + r0, jnp.zeros((16,), jnp.int32)])
            for e in range(16):
                v = xs[r0 + e, pl.ds(0, 16)] * xd[r0 + e, pl.ds(0, 16)]
                for dd in range(1, D // 16):
                    v = v + (xs[r0 + e, pl.ds(dd * 16, 16)]
                             * xd[r0 + e, pl.ds(dd * 16, 16)])
                tmp[r0 + e, pl.ds(0, 16)] = v
            dots = plsc.load_gather(
                tmp, [iota16 + r0, jnp.zeros((16,), jnp.int32)])
            for dd in range(1, 16):
                dots = dots + plsc.load_gather(
                    tmp, [iota16 + r0, jnp.full((16,), dd, jnp.int32)])
            wden = jnp.exp(dots * bet)
            wnum = wden * normsrc
            plsc.store_scatter(
                wrow, [iota16 + r0, jnp.zeros((16,), jnp.int32)], wden)
            wnv[pl.ds(r0, 16)] = wnum
            for e in range(16):
                sca = plsc.load_gather(wnv, [jnp.full((16,), e, jnp.int32) + r0])
                for dd in range(D // 16):
                    xd[r0 + e, pl.ds(dd * 16, 16)] = (
                        xs[r0 + e, pl.ds(dd * 16, 16)] * sca)

        cs1 = pltpu.async_copy(xd, num_sh.at[dstv], sems, add=True)
        cs2 = pltpu.async_copy(wrow, den_sh.at[dstv], sems, add=True)
        cs1.wait()
        cs2.wait()

    plsc.subcore_barrier()

    pltpu.sync_copy(num_sh.at[pl.ds(nrow0, RPT)],
                    num_out.at[pl.ds(cidx * N + nrow0, RPT)])
    pltpu.sync_copy(den_sh.at[pl.ds(nrow0, RPT)],
                    den_out.at[pl.ds(cidx * N + nrow0, RPT)])

    @pl.when(sidx == 0)
    def _():
        pltpu.sync_copy(num_sh.at[pl.ds(NS * RPT, TAIL)],
                        num_out.at[pl.ds(cidx * N + NS * RPT, TAIL)])
        pltpu.sync_copy(den_sh.at[pl.ds(NS * RPT, TAIL)],
                        den_out.at[pl.ds(cidx * N + NS * RPT, TAIL)])


def _edge_stage(xn, norm2d, srci, dsti, betav):
    mesh = plsc.VectorSubcoreMesh(core_axis_name="c", subcore_axis_name="s")
    cp = pltpu.CompilerParams()
    if "needs_layout_passes" in pltpu.CompilerParams.__dataclass_fields__:
        cp = dataclasses.replace(cp, needs_layout_passes=False)
    if "use_tc_tiling_on_sc" in pltpu.CompilerParams.__dataclass_fields__:
        cp = dataclasses.replace(cp, use_tc_tiling_on_sc=False)
    kern = pl.kernel(
        _edge_body,
        compiler_params=cp,
        out_type=(
            jax.ShapeDtypeStruct((NC * N, D), jnp.float32),
            jax.ShapeDtypeStruct((NC * N, 16), jnp.float32),
        ),
        mesh=mesh,
        scratch_types=[
            pltpu.VMEM((C, D), jnp.float32),
            pltpu.VMEM((C, D), jnp.float32),
            pltpu.VMEM((C, 16), jnp.float32),
            pltpu.VMEM((C, 16), jnp.float32),
            pltpu.VMEM((C,), jnp.int32),
            pltpu.VMEM((C,), jnp.int32),
            pltpu.VMEM((C, 16), jnp.float32),
            pltpu.VMEM((16,), jnp.float32),
            pltpu.VMEM((C,), jnp.float32),
            pltpu.VMEM_SHARED((N, D), jnp.float32),
            pltpu.VMEM_SHARED((N, 16), jnp.float32),
            pltpu.SemaphoreType.DMA,
            pltpu.SemaphoreType.DMA,
            pltpu.SemaphoreType.DMA,
        ],
    )
    return kern(xn, norm2d, srci, dsti, betav)


def _lstm_body(num_ref, den_ref, sw_ref, x_ref, h_ref, c_ref,
               wih_ref, whh_ref, h1_ref, c1_ref):
    sw = sw_ref[...]
    num = num_ref[0] + num_ref[1] + sw * x_ref[...]
    den = den_ref[0, :, 0:1] + den_ref[1, :, 0:1] + sw
    xb = jnp.tanh(num / jnp.maximum(den, 1e-16))
    dn = (((1,), (1,)), ((), ()))
    gates = lax.dot_general(xb, wih_ref[...], dn,
                            preferred_element_type=jnp.float32,
                            precision=lax.Precision.HIGHEST)
    gates = gates + lax.dot_general(h_ref[...], whh_ref[...], dn,
                                    preferred_element_type=jnp.float32,
                                    precision=lax.Precision.HIGHEST)
    ii = jax.nn.sigmoid(gates[:, 0:D])
    ff = jax.nn.sigmoid(gates[:, D:2 * D])
    gg = jnp.tanh(gates[:, 2 * D:3 * D])
    oo = jax.nn.sigmoid(gates[:, 3 * D:4 * D])
    c1 = ff * c_ref[...] + ii * gg
    h1_ref[...] = oo * jnp.tanh(c1)
    c1_ref[...] = c1


def _lstm_stage(num2, den2, sw, x, h0, c0, W_ih, W_hh):
    nb = 10
    blk = N // nb
    return pl.pallas_call(
        _lstm_body,
        grid=(nb,),
        in_specs=[
            pl.BlockSpec((2, blk, D), lambda i: (0, i, 0)),
            pl.BlockSpec((2, blk, 16), lambda i: (0, i, 0)),
            pl.BlockSpec((blk, 1), lambda i: (i, 0)),
            pl.BlockSpec((blk, D), lambda i: (i, 0)),
            pl.BlockSpec((blk, D), lambda i: (i, 0)),
            pl.BlockSpec((blk, D), lambda i: (i, 0)),
            pl.BlockSpec((4 * D, D), lambda i: (0, 0)),
            pl.BlockSpec((4 * D, D), lambda i: (0, 0)),
        ],
        out_specs=[
            pl.BlockSpec((blk, D), lambda i: (i, 0)),
            pl.BlockSpec((blk, D), lambda i: (i, 0)),
        ],
        out_shape=(
            jax.ShapeDtypeStruct((N, D), jnp.float32),
            jax.ShapeDtypeStruct((N, D), jnp.float32),
        ),
    )(num2, den2, sw, x, h0, c0, W_ih, W_hh)


def kernel(x, edge_index, h, c, beta, W_ih, W_hh):
    beta2d = jnp.reshape(beta.astype(jnp.float32), (1, 1))
    xn, normv, sw = _prep(x, beta2d)
    norm2d = jnp.broadcast_to(normv, (N, 16))

    betav = jnp.broadcast_to(jnp.reshape(beta.astype(jnp.float32), (1,)), (16,))
    num2, den2 = _edge_stage(xn, norm2d, edge_index[0], edge_index[1], betav)

    h1, c1 = _lstm_stage(
        jnp.reshape(num2, (NC, N, D)), jnp.reshape(den2, (NC, N, 16)),
        sw, x, h[0], c[0], W_ih, W_hh)
    return (h1, h1[None, :, :], c1[None, :, :])
